# baseline (speedup 1.0000x reference)
import jax
import jax.numpy as jnp
from jax import lax
from jax.experimental import pallas as pl
from jax.experimental.pallas import tpu as pltpu

_T, _N = 8, 1000
_NP = 1024
_LINK_TH = 0.2
_IOU_TH = 0.2
_MAX_SEQ = 50
_NEG = -1e9
_BIG = 2 ** 30


def _seqnms_body(x1_ref, y1_ref, x2_ref, y2_ref, cls_ref, sc_ref,
                 x1t_ref, y1t_ref, x2t_ref, y2t_ref, clst_ref,
                 out_ref,
                 pen_ref, alive_ref, dps_ref):
    out_ref[...] = sc_ref[...]
    col2d = lax.broadcasted_iota(jnp.int32, (_T, _NP), 1)
    alive_ref[...] = (col2d < _N).astype(jnp.float32)

    lane = lax.broadcasted_iota(jnp.int32, (1, _NP), 1)
    t_col = lax.broadcasted_iota(jnp.int32, (_T, 1), 0)
    lane2d = lax.broadcasted_iota(jnp.int32, (_T, _NP), 1)

    for t in range(_T - 1):
        bx1 = x1t_ref[:, t:t + 1]
        by1 = y1t_ref[:, t:t + 1]
        bx2 = x2t_ref[:, t:t + 1]
        by2 = y2t_ref[:, t:t + 1]
        ax1 = x1_ref[t + 1:t + 2, :]
        ay1 = y1_ref[t + 1:t + 2, :]
        ax2 = x2_ref[t + 1:t + 2, :]
        ay2 = y2_ref[t + 1:t + 2, :]
        ix1 = jnp.maximum(bx1, ax1)
        iy1 = jnp.maximum(by1, ay1)
        ix2 = jnp.minimum(bx2, ax2)
        iy2 = jnp.minimum(by2, ay2)
        inter = jnp.maximum(ix2 - ix1, 0.0) * jnp.maximum(iy2 - iy1, 0.0)
        area_a = jnp.maximum(bx2 - bx1, 0.0) * jnp.maximum(by2 - by1, 0.0)
        area_b = jnp.maximum(ax2 - ax1, 0.0) * jnp.maximum(ay2 - ay1, 0.0)
        union = area_a + area_b - inter
        iou = inter / jnp.maximum(union, 1e-8)
        cls_eq = clst_ref[:, t:t + 1] == cls_ref[t + 1:t + 2, :]
        linkb = (iou >= _LINK_TH) & cls_eq
        pen_ref[t] = linkb.astype(jnp.float32) * 1e9 - 1e9

    def iter_body(_, carry):
        alive_last = alive_ref[_T - 1:_T, :] > 0.0
        dps_ref[_T - 1:_T, :] = jnp.where(
            alive_last, out_ref[_T - 1:_T, :], _NEG)
        for t in range(_T - 2, -1, -1):
            nxt_row = jnp.where(alive_ref[t + 1:t + 2, :] > 0.0,
                                dps_ref[t + 1:t + 2, :], _NEG)
            cand = pen_ref[t] + nxt_row
            best = jnp.max(cand, axis=1, keepdims=True)
            ext = jnp.maximum(best, 0.0)
            dps_ref[t:t + 1, :] = jnp.where(
                alive_ref[t:t + 1, :] > 0.0,
                out_ref[t:t + 1, :], _NEG) + ext.T

        dp = dps_ref[...]
        best_val = jnp.max(dp)
        row_max = jnp.max(dp, axis=1, keepdims=True)
        t0 = jnp.min(jnp.where(row_max == best_val, t_col, _BIG))
        row_iota = lax.broadcasted_iota(jnp.int32, (_T, _NP), 0)
        i0 = jnp.min(jnp.where((dp == best_val) & (row_iota == t0),
                               lane2d, _BIG))
        active = best_val > 0.0

        in_seq = jnp.zeros((), jnp.bool_)
        cur_i = jnp.zeros((), jnp.int32)
        mem_col = jnp.zeros((_T, 1), jnp.float32)
        idx_col = jnp.zeros((_T, 1), jnp.int32)
        for t in range(_T):
            if t > 0:
                onehot = (lane == cur_i).astype(jnp.float32)
                prow = jax.lax.dot_general(
                    onehot, pen_ref[t - 1],
                    dimension_numbers=(((1,), (0,)), ((), ())),
                    preferred_element_type=jnp.float32)
                nxtm = jnp.where(alive_ref[t:t + 1, :] > 0.0,
                                 dps_ref[t:t + 1, :], _NEG)
                crow = prow + nxtm
                bestr = jnp.max(crow)
                nxt_i = jnp.min(jnp.where(crow == bestr, lane, _BIG))
                nxt_i = jnp.where(bestr > 0.0, nxt_i, -1)
                cont = in_seq & (nxt_i >= 0)
                cur_i = jnp.where(cont, nxt_i, cur_i)
                in_seq = cont
            start = t0 == t
            in_seq = in_seq | start
            cur_i = jnp.where(start, i0, cur_i)
            m_f = (in_seq & active).astype(jnp.float32)
            mem_col = jnp.where(t_col == t, m_f, mem_col)
            idx_col = jnp.where(t_col == t, cur_i, idx_col)

        is_ii = lane2d == idx_col
        svals = jnp.sum(jnp.where(is_ii, out_ref[...], 0.0),
                        axis=1, keepdims=True)
        seq_sum = jnp.sum(mem_col * svals)
        seq_cnt = jnp.sum(mem_col)
        avg = seq_sum / jnp.maximum(seq_cnt, 1.0)

        mem_b = mem_col > 0.0
        hit = mem_b & is_ii
        out_ref[...] = jnp.where(hit, avg, out_ref[...])
        arow = alive_ref[...] > 0.0
        alive_t = arow & ~hit
        bx1 = jnp.sum(jnp.where(is_ii, x1_ref[...], 0.0),
                      axis=1, keepdims=True)
        by1 = jnp.sum(jnp.where(is_ii, y1_ref[...], 0.0),
                      axis=1, keepdims=True)
        bx2 = jnp.sum(jnp.where(is_ii, x2_ref[...], 0.0),
                      axis=1, keepdims=True)
        by2 = jnp.sum(jnp.where(is_ii, y2_ref[...], 0.0),
                      axis=1, keepdims=True)
        ix1 = jnp.maximum(bx1, x1_ref[...])
        iy1 = jnp.maximum(by1, y1_ref[...])
        ix2 = jnp.minimum(bx2, x2_ref[...])
        iy2 = jnp.minimum(by2, y2_ref[...])
        inter = jnp.maximum(ix2 - ix1, 0.0) * jnp.maximum(iy2 - iy1, 0.0)
        area_a = jnp.maximum(bx2 - bx1, 0.0) * jnp.maximum(by2 - by1, 0.0)
        area_b = (jnp.maximum(x2_ref[...] - x1_ref[...], 0.0)
                  * jnp.maximum(y2_ref[...] - y1_ref[...], 0.0))
        union = area_a + area_b - inter
        iou = inter / jnp.maximum(union, 1e-8)
        cls_ii = jnp.sum(jnp.where(is_ii, cls_ref[...], 0),
                         axis=1, keepdims=True)
        sup = (iou >= _IOU_TH) & (cls_ref[...] == cls_ii) & alive_t
        alive_f = (alive_t & ~sup).astype(jnp.float32)
        alive_ref[...] = jnp.where(mem_b, alive_f, alive_ref[...])
        return carry

    lax.fori_loop(0, _MAX_SEQ, iter_body, 0)


@jax.jit
def kernel(boxes, scores, classes):
    classes = classes.astype(jnp.int32)
    pad = _NP - _N
    x1 = jnp.pad(boxes[:, :, 0], ((0, 0), (0, pad)))
    y1 = jnp.pad(boxes[:, :, 1], ((0, 0), (0, pad)))
    x2 = jnp.pad(boxes[:, :, 2], ((0, 0), (0, pad)))
    y2 = jnp.pad(boxes[:, :, 3], ((0, 0), (0, pad)))
    sc = jnp.pad(scores, ((0, 0), (0, pad)))
    cls = jnp.pad(classes, ((0, 0), (0, pad)), constant_values=-1)
    out = pl.pallas_call(
        _seqnms_body,
        out_shape=jax.ShapeDtypeStruct((_T, _NP), jnp.float32),
        scratch_shapes=[
            pltpu.VMEM((_T - 1, _NP, _NP), jnp.float32),
            pltpu.VMEM((_T, _NP), jnp.float32),
            pltpu.VMEM((_T, _NP), jnp.float32),
        ],
        compiler_params=pltpu.CompilerParams(
            vmem_limit_bytes=100 * 1024 * 1024),
    )(x1, y1, x2, y2, cls, sc,
      x1.T, y1.T, x2.T, y2.T, cls.T)
    return out[:, :_N]


# speedup vs baseline: 1.3015x; 1.3015x over previous
import jax
import jax.numpy as jnp
from jax import lax
from jax.experimental import pallas as pl
from jax.experimental.pallas import tpu as pltpu

_T, _N = 8, 1000
_NP = 1024
_LINK_TH = 0.2
_IOU_TH = 0.2
_MAX_SEQ = 50
_NEG = -1e9
_BIG = 2 ** 30


def _seqnms_body(x1_ref, y1_ref, x2_ref, y2_ref, cls_ref, sc_ref,
                 x1t_ref, y1t_ref, x2t_ref, y2t_ref, clst_ref,
                 out_ref,
                 pen_ref, alive_ref, dps_ref, ext_ref):
    out_ref[...] = sc_ref[...]
    col2d = lax.broadcasted_iota(jnp.int32, (_T, _NP), 1)
    alive_ref[...] = (col2d < _N).astype(jnp.float32)

    lane = lax.broadcasted_iota(jnp.int32, (1, _NP), 1)
    t_col = lax.broadcasted_iota(jnp.int32, (_T, 1), 0)
    lane2d = lax.broadcasted_iota(jnp.int32, (_T, _NP), 1)

    for t in range(_T - 1):
        bx1 = x1t_ref[:, t:t + 1]
        by1 = y1t_ref[:, t:t + 1]
        bx2 = x2t_ref[:, t:t + 1]
        by2 = y2t_ref[:, t:t + 1]
        ax1 = x1_ref[t + 1:t + 2, :]
        ay1 = y1_ref[t + 1:t + 2, :]
        ax2 = x2_ref[t + 1:t + 2, :]
        ay2 = y2_ref[t + 1:t + 2, :]
        ix1 = jnp.maximum(bx1, ax1)
        iy1 = jnp.maximum(by1, ay1)
        ix2 = jnp.minimum(bx2, ax2)
        iy2 = jnp.minimum(by2, ay2)
        inter = jnp.maximum(ix2 - ix1, 0.0) * jnp.maximum(iy2 - iy1, 0.0)
        area_a = jnp.maximum(bx2 - bx1, 0.0) * jnp.maximum(by2 - by1, 0.0)
        area_b = jnp.maximum(ax2 - ax1, 0.0) * jnp.maximum(ay2 - ay1, 0.0)
        union = area_a + area_b - inter
        iou = inter / jnp.maximum(union, 1e-8)
        cls_eq = clst_ref[:, t:t + 1] == cls_ref[t + 1:t + 2, :]
        linkb = (iou >= _LINK_TH) & cls_eq
        pen_ref[t] = linkb.astype(jnp.float32) * 1e9 - 1e9

    def iter_body(_, carry):
        alive_last = alive_ref[_T - 1:_T, :] > 0.0
        dps_ref[_T - 1:_T, :] = jnp.where(
            alive_last, out_ref[_T - 1:_T, :], _NEG)
        for t in range(_T - 2, -1, -1):
            cand = pen_ref[t] + dps_ref[t + 1:t + 2, :]
            best = jnp.max(cand, axis=1, keepdims=True)
            ext = jnp.maximum(best, 0.0).T
            ext_ref[t:t + 1, :] = ext
            dps_ref[t:t + 1, :] = jnp.where(
                alive_ref[t:t + 1, :] > 0.0,
                out_ref[t:t + 1, :], _NEG) + ext

        dp = dps_ref[...]
        best_val = jnp.max(dp)
        row_max = jnp.max(dp, axis=1, keepdims=True)
        t0 = jnp.min(jnp.where(row_max == best_val, t_col, _BIG))
        row_iota = lax.broadcasted_iota(jnp.int32, (_T, _NP), 0)
        i0 = jnp.min(jnp.where((dp == best_val) & (row_iota == t0),
                               lane2d, _BIG))
        active = best_val > 0.0

        in_seq = jnp.zeros((), jnp.bool_)
        cur_i = jnp.zeros((), jnp.int32)
        mem_col = jnp.zeros((_T, 1), jnp.float32)
        idx_col = jnp.zeros((_T, 1), jnp.int32)
        for t in range(_T):
            if t > 0:
                sel = lane == cur_i
                prev = slice(t - 1, t)
                ext_ci = jnp.sum(jnp.where(sel, ext_ref[prev, :], 0.0))
                bx1 = jnp.sum(jnp.where(sel, x1_ref[prev, :], 0.0))
                by1 = jnp.sum(jnp.where(sel, y1_ref[prev, :], 0.0))
                bx2 = jnp.sum(jnp.where(sel, x2_ref[prev, :], 0.0))
                by2 = jnp.sum(jnp.where(sel, y2_ref[prev, :], 0.0))
                cls_ci = jnp.sum(jnp.where(sel, cls_ref[prev, :], 0))
                ax1 = x1_ref[t:t + 1, :]
                ay1 = y1_ref[t:t + 1, :]
                ax2 = x2_ref[t:t + 1, :]
                ay2 = y2_ref[t:t + 1, :]
                ix1 = jnp.maximum(bx1, ax1)
                iy1 = jnp.maximum(by1, ay1)
                ix2 = jnp.minimum(bx2, ax2)
                iy2 = jnp.minimum(by2, ay2)
                inter = (jnp.maximum(ix2 - ix1, 0.0)
                         * jnp.maximum(iy2 - iy1, 0.0))
                area_a = (jnp.maximum(bx2 - bx1, 0.0)
                          * jnp.maximum(by2 - by1, 0.0))
                area_b = (jnp.maximum(ax2 - ax1, 0.0)
                          * jnp.maximum(ay2 - ay1, 0.0))
                union = area_a + area_b - inter
                iou = inter / jnp.maximum(union, 1e-8)
                linkr = (iou >= _LINK_TH) & (cls_ref[t:t + 1, :] == cls_ci)
                hitj = linkr & (dps_ref[t:t + 1, :] == ext_ci)
                nxt_i = jnp.min(jnp.where(hitj, lane, _BIG))
                nxt_i = jnp.where(ext_ci > 0.0, nxt_i, -1)
                cont = in_seq & (nxt_i >= 0)
                cur_i = jnp.where(cont, nxt_i, cur_i)
                in_seq = cont
            start = t0 == t
            in_seq = in_seq | start
            cur_i = jnp.where(start, i0, cur_i)
            m_f = (in_seq & active).astype(jnp.float32)
            mem_col = jnp.where(t_col == t, m_f, mem_col)
            idx_col = jnp.where(t_col == t, cur_i, idx_col)

        is_ii = lane2d == idx_col
        svals = jnp.sum(jnp.where(is_ii, out_ref[...], 0.0),
                        axis=1, keepdims=True)
        seq_sum = jnp.sum(mem_col * svals)
        seq_cnt = jnp.sum(mem_col)
        avg = seq_sum / jnp.maximum(seq_cnt, 1.0)

        mem_b = mem_col > 0.0
        hit = mem_b & is_ii
        out_ref[...] = jnp.where(hit, avg, out_ref[...])
        arow = alive_ref[...] > 0.0
        alive_t = arow & ~hit
        bx1 = jnp.sum(jnp.where(is_ii, x1_ref[...], 0.0),
                      axis=1, keepdims=True)
        by1 = jnp.sum(jnp.where(is_ii, y1_ref[...], 0.0),
                      axis=1, keepdims=True)
        bx2 = jnp.sum(jnp.where(is_ii, x2_ref[...], 0.0),
                      axis=1, keepdims=True)
        by2 = jnp.sum(jnp.where(is_ii, y2_ref[...], 0.0),
                      axis=1, keepdims=True)
        ix1 = jnp.maximum(bx1, x1_ref[...])
        iy1 = jnp.maximum(by1, y1_ref[...])
        ix2 = jnp.minimum(bx2, x2_ref[...])
        iy2 = jnp.minimum(by2, y2_ref[...])
        inter = jnp.maximum(ix2 - ix1, 0.0) * jnp.maximum(iy2 - iy1, 0.0)
        area_a = jnp.maximum(bx2 - bx1, 0.0) * jnp.maximum(by2 - by1, 0.0)
        area_b = (jnp.maximum(x2_ref[...] - x1_ref[...], 0.0)
                  * jnp.maximum(y2_ref[...] - y1_ref[...], 0.0))
        union = area_a + area_b - inter
        iou = inter / jnp.maximum(union, 1e-8)
        cls_ii = jnp.sum(jnp.where(is_ii, cls_ref[...], 0),
                         axis=1, keepdims=True)
        sup = (iou >= _IOU_TH) & (cls_ref[...] == cls_ii) & alive_t
        alive_f = (alive_t & ~sup).astype(jnp.float32)
        alive_ref[...] = jnp.where(mem_b, alive_f, alive_ref[...])
        return carry

    lax.fori_loop(0, _MAX_SEQ, iter_body, 0)


@jax.jit
def kernel(boxes, scores, classes):
    classes = classes.astype(jnp.int32)
    pad = _NP - _N
    x1 = jnp.pad(boxes[:, :, 0], ((0, 0), (0, pad)))
    y1 = jnp.pad(boxes[:, :, 1], ((0, 0), (0, pad)))
    x2 = jnp.pad(boxes[:, :, 2], ((0, 0), (0, pad)))
    y2 = jnp.pad(boxes[:, :, 3], ((0, 0), (0, pad)))
    sc = jnp.pad(scores, ((0, 0), (0, pad)))
    cls = jnp.pad(classes, ((0, 0), (0, pad)), constant_values=-1)
    out = pl.pallas_call(
        _seqnms_body,
        out_shape=jax.ShapeDtypeStruct((_T, _NP), jnp.float32),
        scratch_shapes=[
            pltpu.VMEM((_T - 1, _NP, _NP), jnp.float32),
            pltpu.VMEM((_T, _NP), jnp.float32),
            pltpu.VMEM((_T, _NP), jnp.float32),
            pltpu.VMEM((_T - 1, _NP), jnp.float32),
        ],
        compiler_params=pltpu.CompilerParams(
            vmem_limit_bytes=100 * 1024 * 1024),
    )(x1, y1, x2, y2, cls, sc,
      x1.T, y1.T, x2.T, y2.T, cls.T)
    return out[:, :_N]
